# baseline (speedup 1.0000x reference)
import jax
import jax.numpy as jnp
from jax import lax
from jax.experimental import pallas as pl
from jax.experimental.pallas import tpu as pltpu
from jax.experimental.pallas import tpu_sc as plsc

N = 10000
NP = 10240
E = 320000
D = 128
NC = 2
NS = 16
L = 16
K = 80
NB = 128
EPAD = NC * NS * NB * K
NBT = EPAD // K
W = 3 * K
CH = 8
CN = NB // CH
PAIRS = NB // 2
RPT = NP // NS
ZR = 16


def _body(edata_h, b_h, out0_h, out1_h,
          ebuf, rsh0, rsh1, gbuf0, gbuf1, sbuf0, sbuf1, zbuf, acc,
          semg0, semg1, sems0, sems1, semc, zsem):
    rsh = [rsh0, rsh1]
    gbuf = [gbuf0, gbuf1]
    sbuf = [sbuf0, sbuf1]
    semg = [semg0, semg1]
    sems = [sems0, sems1]

    c = lax.axis_index("c")
    s = lax.axis_index("s")
    mbase = (c * NS + s) * NB
    rbase = s * RPT

    def fire_chunk(g):
        pltpu.async_copy(edata_h.at[pl.ds(mbase + g * CH, CH)],
                         ebuf.at[g % 2], semc)

    def wait_chunk():
        pltpu.make_async_copy(edata_h.at[pl.ds(0, CH)], ebuf.at[0],
                              semc).wait()

    def fire_gather(b, j):
        pltpu.async_copy(
            b_h.at[ebuf.at[0, 0, pl.ds(0, K)]],
            gbuf[b], semg[b])

    def wait_gather(b):
        pltpu.make_async_copy(b_h.at[rsh[b]], gbuf[b], semg[b]).wait()

    def snapshot(b, j):
        cs = (j // CH) % 2
        bi = j % CH
        vvecs = []
        for t in range(K // L):
            sl = pl.ds(t * L, L)
            rsh[b][sl] = ebuf[0, 0, pl.ds(K + t * L, L)]
            vvecs.append(zbuf[0, pl.ds(0, L)])
        return vvecs

    def scale(b, vvecs):
        for t in range(K // L):
            for u in range(L):
                i = t * L + u
                v = vvecs[t][u]
                for q in range(D // L):
                    sl = pl.ds(q * L, L)
                    sbuf[b][i, sl] = gbuf[b][i, sl] * v

    def fire_scatter(b):
        pltpu.async_copy(sbuf[b], acc.at[rsh[b]], sems[b], add=True)

    def wait_scatter(b):
        pltpu.make_async_copy(sbuf[b], acc.at[rsh[b]], sems[b]).wait()

    zero = jnp.zeros((L,), jnp.float32)

    def zrow(i, carry):
        for q in range(D // L):
            zbuf[i, pl.ds(q * L, L)] = zero
        return carry

    lax.fori_loop(0, ZR, zrow, 0)
    for z in range(RPT // ZR):
        pltpu.async_copy(zbuf, acc.at[pl.ds(rbase + z * ZR, ZR)], zsem)
    for z in range(RPT // ZR):
        pltpu.make_async_copy(zbuf, acc.at[pl.ds(rbase + z * ZR, ZR)],
                              zsem).wait()
    plsc.subcore_barrier()

    fire_chunk(0)
    wait_chunk()
    fire_gather(0, 0)
    fire_chunk(1)

    def pair(j2, carry):
        for b in range(2):
            j = 2 * j2 + b
            wait_gather(b)
            if b == 0:
                fire_gather(1, j + 1)
            else:
                @pl.when(j2 < PAIRS - 1)
                def _():
                    @pl.when(j % CH == CH - 1)
                    def _():
                        wait_chunk()
                    fire_gather(0, j + 1)

            @pl.when(j2 >= 1)
            def _():
                wait_scatter(b)

            vvecs = snapshot(b, j)
            if b == 1:
                @pl.when((j % CH == CH - 1) & (j // CH < CN - 2))
                def _():
                    fire_chunk(j // CH + 2)
            scale(b, vvecs)
            fire_scatter(b)
        return carry

    lax.fori_loop(0, PAIRS, pair, 0)

    wait_scatter(0)
    wait_scatter(1)
    plsc.subcore_barrier()

    osl = pl.ds(rbase, RPT)

    @pl.when(c == 0)
    def _():
        pltpu.sync_copy(acc.at[osl], out0_h.at[osl])

    @pl.when(c == 1)
    def _():
        pltpu.sync_copy(acc.at[osl], out1_h.at[osl])


_spmm = pl.kernel(
    _body,
    out_type=(jax.ShapeDtypeStruct((NP, D), jnp.float32),
              jax.ShapeDtypeStruct((NP, D), jnp.float32)),
    mesh=plsc.VectorSubcoreMesh(
        core_axis_name="c", subcore_axis_name="s",
        num_cores=NC, num_subcores=NS),
    scratch_types=[
        pltpu.VMEM((2, CH, W), jnp.int32),
        pltpu.VMEM((K,), jnp.int32),
        pltpu.VMEM((K,), jnp.int32),
        pltpu.VMEM((K, D), jnp.float32),
        pltpu.VMEM((K, D), jnp.float32),
        pltpu.VMEM((K, D), jnp.float32),
        pltpu.VMEM((K, D), jnp.float32),
        pltpu.VMEM((ZR, D), jnp.float32),
        pltpu.VMEM_SHARED((NP, D), jnp.float32),
        pltpu.SemaphoreType.DMA,
        pltpu.SemaphoreType.DMA,
        pltpu.SemaphoreType.DMA,
        pltpu.SemaphoreType.DMA,
        pltpu.SemaphoreType.DMA,
        pltpu.SemaphoreType.DMA,
    ],
)


def _add_body(a_ref, b_ref, o_ref):
    sl = pl.ds(0, N)
    o_ref[...] = a_ref[sl, :] + b_ref[sl, :]


_combine = pl.pallas_call(
    _add_body,
    out_shape=jax.ShapeDtypeStruct((N, D), jnp.float32),
)


def kernel(edge_index, values, B):
    rows = edge_index[0]
    cols = edge_index[1]
    pc = jnp.pad(cols, (0, EPAD - E)).reshape(NBT, K)
    pr = jnp.pad(rows, (0, EPAD - E)).reshape(NBT, K)
    pv = lax.bitcast_convert_type(
        jnp.pad(values, (0, EPAD - E)), jnp.int32).reshape(NBT, K)
    edata = jnp.concatenate([pc, pr, pv], axis=1)
    p0, p1 = _spmm(edata, B)
    return _combine(p0, p1)


# speedup vs baseline: 3.0433x; 3.0433x over previous
import jax
import jax.numpy as jnp
from jax import lax
from jax.experimental import pallas as pl
from jax.experimental.pallas import tpu as pltpu
from jax.experimental.pallas import tpu_sc as plsc

N = 10000
NP = 10240
E = 320000
D = 128
NC = 2
NS = 16
L = 16
EPT = E // (NC * NS)
K = 80
W = 3 * K
NB = EPT // K
PAIRS = NB // 2
RPT = NP // NS
ZR = 32


def _body(edata_h, b_h, out0_h, out1_h,
          ebuf0, ebuf1, rsh0, rsh1,
          gbuf0, gbuf1, sbuf0, sbuf1, zbuf, acc,
          semg0, semg1, sems0, sems1, semi0, semi1, zsem):
    ebuf = [ebuf0, ebuf1]
    rsh = [rsh0, rsh1]
    gbuf = [gbuf0, gbuf1]
    sbuf = [sbuf0, sbuf1]
    semg = [semg0, semg1]
    sems = [sems0, sems1]
    semi = [semi0, semi1]

    c = lax.axis_index("c")
    s = lax.axis_index("s")
    mbase = (c * NS + s) * NB
    rbase = s * RPT

    def stage(j, b):
        pltpu.async_copy(edata_h.at[pl.ds((mbase + j) * W, W)],
                         ebuf[b], semi[b])

    def wait_stage(b):
        pltpu.make_async_copy(edata_h.at[pl.ds(0, W)], ebuf[b],
                              semi[b]).wait()

    def fire_gather(b):
        pltpu.async_copy(b_h.at[ebuf[b].at[pl.ds(0, K)]], gbuf[b], semg[b])

    def wait_gather(b):
        pltpu.make_async_copy(b_h.at[rsh[b]], gbuf[b], semg[b]).wait()

    def snapshot(b):
        vvecs = []
        for t in range(K // L):
            sl = pl.ds(t * L, L)
            rsh[b][sl] = ebuf[b][pl.ds(K + t * L, L)]
            vvecs.append(
                lax.bitcast_convert_type(
                    ebuf[b][pl.ds(2 * K + t * L, L)], jnp.float32))
        return vvecs

    def scale(b, vvecs):
        for t in range(K // L):
            for u in range(L):
                i = t * L + u
                v = vvecs[t][u]
                for q in range(D // L):
                    sl = pl.ds(q * L, L)
                    sbuf[b][i, sl] = gbuf[b][i, sl] * v

    def fire_scatter(b):
        pltpu.async_copy(sbuf[b], acc.at[rsh[b]], sems[b], add=True)

    def wait_scatter(b):
        pltpu.make_async_copy(sbuf[b], acc.at[rsh[b]], sems[b]).wait()

    zero = jnp.zeros((L,), jnp.float32)

    def zrow(i, carry):
        for q in range(D // L):
            zbuf[i, pl.ds(q * L, L)] = zero
        return carry

    lax.fori_loop(0, ZR, zrow, 0)
    for z in range(RPT // ZR):
        pltpu.async_copy(zbuf, acc.at[pl.ds(rbase + z * ZR, ZR)], zsem)
    for z in range(RPT // ZR):
        pltpu.make_async_copy(zbuf, acc.at[pl.ds(rbase + z * ZR, ZR)],
                              zsem).wait()
    plsc.subcore_barrier()

    stage(0, 0)
    wait_stage(0)
    fire_gather(0)
    stage(1, 1)

    def pair(j2, carry):
        for b in range(2):
            j = 2 * j2 + b
            wait_gather(b)
            wait_stage(1 - b)
            fire_gather(1 - b)

            @pl.when(j2 >= 1)
            def _():
                wait_scatter(b)

            vvecs = snapshot(b)
            if b == 0:
                stage(j + 2, 0)
            else:
                @pl.when(j2 < PAIRS - 1)
                def _():
                    stage(j + 2, 1)
            scale(b, vvecs)
            fire_scatter(b)
        return carry

    lax.fori_loop(0, PAIRS, pair, 0)

    wait_gather(0)
    wait_scatter(0)
    vvecs = snapshot(0)
    scale(0, vvecs)
    fire_scatter(0)
    wait_scatter(0)
    wait_scatter(1)
    plsc.subcore_barrier()

    osl = pl.ds(rbase, RPT)

    @pl.when(c == 0)
    def _():
        pltpu.sync_copy(acc.at[osl], out0_h.at[osl])

    @pl.when(c == 1)
    def _():
        pltpu.sync_copy(acc.at[osl], out1_h.at[osl])


_spmm = pl.kernel(
    _body,
    out_type=(jax.ShapeDtypeStruct((NP, D), jnp.float32),
              jax.ShapeDtypeStruct((NP, D), jnp.float32)),
    mesh=plsc.VectorSubcoreMesh(
        core_axis_name="c", subcore_axis_name="s",
        num_cores=NC, num_subcores=NS),
    scratch_types=[
        pltpu.VMEM((W,), jnp.int32),
        pltpu.VMEM((W,), jnp.int32),
        pltpu.VMEM((K,), jnp.int32),
        pltpu.VMEM((K,), jnp.int32),
        pltpu.VMEM((K, D), jnp.float32),
        pltpu.VMEM((K, D), jnp.float32),
        pltpu.VMEM((K, D), jnp.float32),
        pltpu.VMEM((K, D), jnp.float32),
        pltpu.VMEM((ZR, D), jnp.float32),
        pltpu.VMEM_SHARED((NP, D), jnp.float32),
        pltpu.SemaphoreType.DMA,
        pltpu.SemaphoreType.DMA,
        pltpu.SemaphoreType.DMA,
        pltpu.SemaphoreType.DMA,
        pltpu.SemaphoreType.DMA,
        pltpu.SemaphoreType.DMA,
        pltpu.SemaphoreType.DMA,
    ],
)


def _add_body(a_ref, b_ref, o_ref):
    sl = pl.ds(0, N)
    o_ref[...] = a_ref[sl, :] + b_ref[sl, :]


_combine = pl.pallas_call(
    _add_body,
    out_shape=jax.ShapeDtypeStruct((N, D), jnp.float32),
)


def kernel(edge_index, values, B):
    colsb = edge_index[1].reshape(E // K, K)
    rowsb = edge_index[0].reshape(E // K, K)
    vb = lax.bitcast_convert_type(values, jnp.int32).reshape(E // K, K)
    edata = jnp.concatenate([colsb, rowsb, vb], axis=1).reshape(-1)
    p0, p1 = _spmm(edata, B)
    return _combine(p0, p1)
